# baseline (speedup 1.0000x reference)
import functools

import jax
import jax.numpy as jnp
from jax import lax
from jax.experimental import pallas as pl
from jax.experimental.pallas import tpu as pltpu
from jax.experimental.pallas import tpu_sc as plsc

_INFO = plsc.get_sparse_core_info()
_NC, _NS = _INFO.num_cores, _INFO.num_subcores
_NW = _NC * _NS

_G = 128
_HALF = 4
_NBUF = 2 * _HALF
_QUARTERS = 4


@functools.lru_cache(maxsize=None)
def _make_lookup(bf: int, vocab2: int, d: int, dpad: int):
    b_per_w = bf // _NW
    n_chunks = b_per_w // _G
    n_body = n_chunks // _NBUF
    mesh = plsc.VectorSubcoreMesh(core_axis_name="c", subcore_axis_name="s")

    @functools.partial(
        pl.kernel,
        mesh=mesh,
        compiler_params=pltpu.CompilerParams(use_tc_tiling_on_sc=False),
        out_type=jax.ShapeDtypeStruct((bf, dpad), jnp.float32),
        scratch_types=[
            pltpu.VMEM((b_per_w,), jnp.int32),
            pltpu.VMEM((_NBUF, _G, d), jnp.float32),
            pltpu.SemaphoreType.DMA((_NBUF,)),
            pltpu.SemaphoreType.DMA((_NBUF,)),
        ],
    )
    def lookup(x_hbm, table_hbm, out_hbm, idx_v, rows_v, gsem, ssem):
        wid = lax.axis_index("s") * _NC + lax.axis_index("c")
        base = wid * b_per_w
        pltpu.sync_copy(x_hbm.at[pl.ds(base, b_per_w)], idx_v)

        def start_gather(c, buf):
            pltpu.make_async_copy(
                table_hbm.at[idx_v.at[pl.ds(c * _G, _G)]],
                rows_v.at[buf],
                gsem.at[buf],
            ).start()

        def wait_gather(buf):
            pltpu.make_async_copy(
                table_hbm.at[pl.ds(0, _G)], rows_v.at[buf], gsem.at[buf]
            ).wait()

        def start_store(c, buf):
            pltpu.make_async_copy(
                rows_v.at[buf],
                out_hbm.at[pl.ds(base + c * _G, _G), pl.ds(0, d)],
                ssem.at[buf],
            ).start()

        def wait_store(buf):
            pltpu.make_async_copy(
                out_hbm.at[pl.ds(0, _G), pl.ds(0, d)],
                rows_v.at[buf],
                ssem.at[buf],
            ).wait()

        for b in range(_HALF):
            start_gather(b, b)

        def body(t, carry):
            c0 = t * _NBUF
            for b in range(_HALF):
                wait_gather(b)
                start_store(c0 + b, b)
            for b in range(_HALF):
                @pl.when(t > 0)
                def _(b=b):
                    wait_store(_HALF + b)
                start_gather(c0 + _HALF + b, _HALF + b)
            for b in range(_HALF):
                wait_gather(_HALF + b)
                start_store(c0 + _HALF + b, _HALF + b)
            for b in range(_HALF):
                @pl.when(t < n_body - 1)
                def _(b=b):
                    wait_store(b)
                    start_gather(c0 + _NBUF + b, b)
            return carry

        lax.fori_loop(0, n_body, body, 0)

        for b in range(_NBUF):
            wait_store(b)

    return lookup


def kernel(x, table):
    vocab, d = table.shape
    q = vocab // _QUARTERS
    tp = jnp.concatenate(
        [
            jnp.pad(lax.slice(table, (i * q, 0), ((i + 1) * q, d)), ((0, 0), (0, d)))
            for i in range(_QUARTERS)
        ],
        axis=0,
    )
    t2 = tp.reshape(2 * vocab, d)
    flat2 = 2 * x.reshape(-1).astype(jnp.int32)
    lookup = _make_lookup(flat2.shape[0], 2 * vocab, d, 2 * d)
    padded = lookup(flat2, t2)
    return padded[:, :d].reshape(x.shape + (d,))


# speedup vs baseline: 2.8862x; 2.8862x over previous
import functools

import jax
import jax.numpy as jnp
from jax import lax
from jax.experimental import pallas as pl
from jax.experimental.pallas import tpu as pltpu
from jax.experimental.pallas import tpu_sc as plsc

_INFO = plsc.get_sparse_core_info()
_NC, _NS = _INFO.num_cores, _INFO.num_subcores
_NW = _NC * _NS

_G = 128
_HALF = 4
_NBUF = 2 * _HALF
_QUARTERS = 4


@functools.lru_cache(maxsize=None)
def _make_lookup(bf: int, vocab2: int, d: int, dpad: int):
    b_per_w = bf // _NW
    n_chunks = b_per_w // _G
    n_body = n_chunks // _NBUF
    mesh = plsc.VectorSubcoreMesh(core_axis_name="c", subcore_axis_name="s")

    @functools.partial(
        pl.kernel,
        mesh=mesh,
        compiler_params=pltpu.CompilerParams(use_tc_tiling_on_sc=False),
        out_type=jax.ShapeDtypeStruct((bf, dpad), jnp.float32),
        scratch_types=[
            pltpu.VMEM((b_per_w,), jnp.int32),
            pltpu.VMEM((_NBUF, _G, d), jnp.float32),
            pltpu.SemaphoreType.DMA((_NBUF,)),
            pltpu.SemaphoreType.DMA((_NBUF,)),
        ],
    )
    def lookup(x_hbm, table_hbm, out_hbm, idx_v, rows_v, gsem, ssem):
        wid = lax.axis_index("s") * _NC + lax.axis_index("c")
        base = wid * b_per_w
        pltpu.sync_copy(x_hbm.at[pl.ds(base, b_per_w)], idx_v)

        def start_gather(c, buf):
            pltpu.make_async_copy(
                table_hbm.at[idx_v.at[pl.ds(c * _G, _G)]],
                rows_v.at[buf],
                gsem.at[buf],
            ).start()

        def wait_gather(buf):
            pltpu.make_async_copy(
                table_hbm.at[pl.ds(0, _G)], rows_v.at[buf], gsem.at[buf]
            ).wait()

        def start_store(c, buf):
            pltpu.make_async_copy(
                rows_v.at[buf],
                out_hbm.at[pl.ds(base + c * _G, _G), pl.ds(0, d)],
                ssem.at[buf],
            ).start()

        def wait_store(buf):
            pltpu.make_async_copy(
                out_hbm.at[pl.ds(0, _G), pl.ds(0, d)],
                rows_v.at[buf],
                ssem.at[buf],
            ).wait()

        for b in range(_HALF):
            start_gather(b, b)

        def body(t, carry):
            c0 = t * _NBUF
            for b in range(_HALF):
                wait_gather(b)
                start_store(c0 + b, b)
            for b in range(_HALF):
                @pl.when(t > 0)
                def _(b=b):
                    wait_store(_HALF + b)
                start_gather(c0 + _HALF + b, _HALF + b)
            for b in range(_HALF):
                wait_gather(_HALF + b)
                start_store(c0 + _HALF + b, _HALF + b)
            for b in range(_HALF):
                @pl.when(t < n_body - 1)
                def _(b=b):
                    wait_store(b)
                    start_gather(c0 + _NBUF + b, b)
            return carry

        lax.fori_loop(0, n_body, body, 0)

        for b in range(_NBUF):
            wait_store(b)

    return lookup


def kernel(x, table):
    vocab, d = table.shape
    tp = jnp.pad(table, ((0, 0), (0, d)))
    t2 = tp.reshape(2 * vocab, d)
    flat2 = 2 * x.reshape(-1).astype(jnp.int32)
    lookup = _make_lookup(flat2.shape[0], 2 * vocab, d, 2 * d)
    padded = lookup(flat2, t2)
    return padded[:, :d].reshape(x.shape + (d,))
